# baseline (speedup 1.0000x reference)
import functools

import jax
import jax.numpy as jnp
from jax import lax
from jax.experimental import pallas as pl
from jax.experimental.pallas import tpu as pltpu
from jax.experimental.pallas import tpu_sc as plsc

VOCAB = 30523
D = 768
NB = 4096
T = 50
TP = 64
NC = 2
NS = 16
NW = NC * NS
BPW = NB // NW
NBUF = 2

_mesh = plsc.VectorSubcoreMesh(core_axis_name="c", subcore_axis_name="s")


@functools.partial(
    pl.kernel,
    mesh=_mesh,
    out_type=jax.ShapeDtypeStruct((NB, TP, D), jnp.float32),
    scratch_types=[
        pltpu.VMEM((BPW * TP,), jnp.int32),
        pltpu.VMEM((NBUF, TP, D), jnp.float32),
        pltpu.SemaphoreType.DMA,
        pltpu.SemaphoreType.DMA,
        pltpu.SemaphoreType.DMA,
        pltpu.SemaphoreType.DMA,
    ],
)
def _emb_lookup(x_hbm, table_hbm, out_hbm, idx_v, rows_v, g0, g1, s0, s1):
    gsem = (g0, g1)
    ssem = (s0, s1)
    wid = lax.axis_index("s") * NC + lax.axis_index("c")
    base = wid * BPW
    pltpu.sync_copy(x_hbm.at[pl.ds(base * TP, BPW * TP)], idx_v)

    def gather_desc(bl, b):
        return pltpu.make_async_copy(
            table_hbm.at[idx_v.at[pl.ds(bl * TP, TP)]], rows_v.at[b], gsem[b]
        )

    def scatter_desc(bl, b):
        return pltpu.make_async_copy(
            rows_v.at[b], out_hbm.at[base + bl], ssem[b]
        )

    for b in range(NBUF):
        gather_desc(b, b).start()

    def body(i, carry):
        bl2 = i * NBUF
        for b in range(NBUF):
            bl = bl2 + b
            gather_desc(bl, b).wait()
            scatter_desc(bl, b).start()

            @pl.when(bl + NBUF < BPW)
            def _():
                scatter_desc(bl, b).wait()
                gather_desc(bl + NBUF, b).start()

        return carry

    lax.fori_loop(0, BPW // NBUF, body, 0, unroll=False)

    for b in range(NBUF):
        scatter_desc(BPW - NBUF + b, b).wait()


def kernel(x, emb_weight):
    xp = jnp.pad(x.astype(jnp.int32), ((0, 0), (0, TP - T)))
    return _emb_lookup(xp.reshape(NB * TP), emb_weight)[:, :T, :]


# speedup vs baseline: 4.8535x; 4.8535x over previous
import functools

import jax
import jax.numpy as jnp
from jax import lax
from jax.experimental import pallas as pl
from jax.experimental.pallas import tpu as pltpu
from jax.experimental.pallas import tpu_sc as plsc

VOCAB = 30523
D = 768
NB = 4096
T = 50
TP = 56
NC = 2
NS = 16
NW = NC * NS
BPW = NB // NW
NBUF = 2

_mesh = plsc.VectorSubcoreMesh(core_axis_name="c", subcore_axis_name="s")


@functools.partial(
    pl.kernel,
    mesh=_mesh,
    out_type=jax.ShapeDtypeStruct((NB, TP, D), jnp.float32),
    scratch_types=[
        pltpu.VMEM((BPW * TP,), jnp.int32),
        pltpu.VMEM((NBUF, TP, D), jnp.float32),
        pltpu.SemaphoreType.DMA,
        pltpu.SemaphoreType.DMA,
        pltpu.SemaphoreType.DMA,
        pltpu.SemaphoreType.DMA,
    ],
)
def _emb_lookup(x_hbm, table_hbm, out_hbm, idx_v, rows_v, g0, g1, s0, s1):
    gsem = (g0, g1)
    ssem = (s0, s1)
    wid = lax.axis_index("s") * NC + lax.axis_index("c")
    base = wid * BPW
    pltpu.sync_copy(x_hbm.at[pl.ds(base * TP, BPW * TP)], idx_v)

    def gather_desc(bl, b):
        return pltpu.make_async_copy(
            table_hbm.at[idx_v.at[pl.ds(bl * TP, TP)]], rows_v.at[b], gsem[b]
        )

    def scatter_desc(bl, b):
        return pltpu.make_async_copy(
            rows_v.at[b], out_hbm.at[base + bl], ssem[b]
        )

    for b in range(NBUF):
        gather_desc(b, b).start()

    def body(i, carry):
        bl2 = i * NBUF
        for b in range(NBUF):
            bl = bl2 + b
            gather_desc(bl, b).wait()
            scatter_desc(bl, b).start()

            @pl.when(bl + NBUF < BPW)
            def _():
                scatter_desc(bl, b).wait()
                gather_desc(bl + NBUF, b).start()

        return carry

    lax.fori_loop(0, BPW // NBUF, body, 0, unroll=False)

    for b in range(NBUF):
        scatter_desc(BPW - NBUF + b, b).wait()


def kernel(x, emb_weight):
    b_iota = jax.lax.broadcasted_iota(jnp.int32, (NB, TP - T), 0)
    t_iota = jax.lax.broadcasted_iota(jnp.int32, (NB, TP - T), 1)
    pad_idx = (b_iota * (TP - T) + t_iota) % VOCAB
    xp = jnp.concatenate([x.astype(jnp.int32), pad_idx], axis=1)
    return _emb_lookup(xp.reshape(NB * TP), emb_weight)[:, :T, :]
